# baseline (speedup 1.0000x reference)
import jax
import jax.numpy as jnp
from jax import lax
from jax.experimental import pallas as pl

BLANK_ID = 0
NUM_CLASSES = 1000
NUM_ROWS = 1024 * 50
PACK = 8
VCOLS = PACK * NUM_CLASSES
VROWS = NUM_ROWS // PACK
VB = 256


def _one_hot_body(ids_ref, sel_ref, out_ref):
    ids = ids_ref[...]
    idm = jnp.where(ids == BLANK_ID, 1023, ids)
    hl = jnp.concatenate([idm >> 5, idm & 31], axis=1).astype(jnp.bfloat16)
    e = jnp.dot(hl, sel_ref[...], preferred_element_type=jnp.float32)
    c = lax.broadcasted_iota(jnp.int32, (1, VCOLS), 1) % NUM_CLASSES
    out_ref[...] = jnp.where(e == c.astype(jnp.float32), 1.0, 0.0).astype(
        jnp.float32
    )


def kernel(outputs, outputs_length):
    ids = outputs.reshape(VROWS, PACK).astype(jnp.int32)
    k = jnp.arange(2 * PACK)[:, None]
    slot = jnp.arange(VCOLS)[None, :] // NUM_CLASSES
    sel = jnp.where(
        k < PACK, (slot == k) * 32.0, (slot == (k - PACK)) * 1.0
    ).astype(jnp.bfloat16)
    out = pl.pallas_call(
        _one_hot_body,
        grid=(VROWS // VB,),
        in_specs=[
            pl.BlockSpec((VB, PACK), lambda i: (i, 0)),
            pl.BlockSpec((2 * PACK, VCOLS), lambda i: (0, 0)),
        ],
        out_specs=pl.BlockSpec((VB, VCOLS), lambda i: (i, 0)),
        out_shape=jax.ShapeDtypeStruct((VROWS, VCOLS), jnp.float32),
    )(ids, sel)
    return out.reshape(1024, 50, NUM_CLASSES), outputs_length


# speedup vs baseline: 1.9649x; 1.9649x over previous
import jax
import jax.numpy as jnp
from jax import lax
from jax.experimental import pallas as pl

BLANK_ID = 0
NUM_CLASSES = 1000
BATCH = 1024
TIME = 50
BB = 32


def _one_hot_body(ids_ref, out_ref):
    ids = ids_ref[...]
    sel = jnp.where(ids == BLANK_ID, -1, ids)
    iota = lax.broadcasted_iota(jnp.int32, out_ref.shape, 2)
    out_ref[...] = (iota == sel).astype(jnp.float32)


def kernel(outputs, outputs_length):
    ids = outputs.reshape(BATCH, TIME, 1).astype(jnp.int32)
    out = pl.pallas_call(
        _one_hot_body,
        grid=(BATCH // BB,),
        in_specs=[pl.BlockSpec((BB, TIME, 1), lambda i: (i, 0, 0))],
        out_specs=pl.BlockSpec((BB, TIME, NUM_CLASSES), lambda i: (i, 0, 0)),
        out_shape=jax.ShapeDtypeStruct((BATCH, TIME, NUM_CLASSES), jnp.float32),
    )(ids)
    return out, outputs_length
